# baseline (speedup 1.0000x reference)
import functools
import math

import jax
import jax.numpy as jnp
from jax import lax
from jax.experimental import pallas as pl
from jax.experimental.pallas import tpu as pltpu
from jax.experimental.pallas import tpu_sc as plsc

_D = 128
_H = 16
_SCALE = math.sqrt(float(_D))
_B = 1024
_L = 50
_NC = 2
_NS = 16
_NW = _NC * _NS
_BT = _B // 8
_QW = 4
_PB = 2
_NP = _QW // _PB


def _scale_body(w_ref, o_ref):
    o_ref[:] = w_ref[:] * _SCALE


_mesh = plsc.VectorSubcoreMesh(core_axis_name="c", subcore_axis_name="s")


@functools.partial(
    pl.kernel,
    out_type=jax.ShapeDtypeStruct((_L, _BT, 128, _D), jnp.float32),
    mesh=_mesh,
    scratch_types=[
        pltpu.VMEM((_L, _QW, 128), jnp.int32),
        pltpu.VMEM((_PB, 128, _D), jnp.float32),
        pltpu.VMEM((_PB, 128, _D), jnp.float32),
        pltpu.VMEM_SHARED((130, _D), jnp.float32),
        pltpu.SemaphoreType.DMA,
        pltpu.SemaphoreType.DMA,
        pltpu.SemaphoreType.DMA,
    ],
    compiler_params=pltpu.CompilerParams(use_tc_tiling_on_sc=True),
)
def _sc_lookup(table_hbm, idx_hbm, out_hbm, idx_v, buf0, buf1, tshared,
               gsem, wsem0, wsem1):
    wid = lax.axis_index("s") * _NC + lax.axis_index("c")
    @pl.when(lax.axis_index("s") == 0)
    def _():
        pltpu.sync_copy(table_hbm, tshared)
    pltpu.sync_copy(idx_hbm.at[wid], idx_v)
    plsc.subcore_barrier()

    bufs = (buf0, buf1)
    wsems = (wsem0, wsem1)
    pending = [None, None]

    steps = [(p, l) for p in range(_NP) for l in range(_L)]

    def start_gathers(step, slot):
        p, l = steps[step]
        return [
            pltpu.async_copy(tshared.at[idx_v.at[l, p * _PB + j]],
                             bufs[slot].at[j], gsem)
            for j in range(_PB)
        ]

    cur_g = start_gathers(0, 0)
    for i, (p, l) in enumerate(steps):
        slot = i % 2
        for g in cur_g:
            g.wait()
        if i + 1 < len(steps):
            nslot = (i + 1) % 2
            if pending[nslot] is not None:
                pending[nslot].wait()
                pending[nslot] = None
            cur_g = start_gathers(i + 1, nslot)
        pending[slot] = pltpu.async_copy(
            bufs[slot],
            out_hbm.at[l, pl.ds(wid * _QW + p * _PB, _PB)],
            wsems[slot])
    for w in pending:
        if w is not None:
            w.wait()


def kernel(inputs, W_v):
    W_s = pl.pallas_call(
        _scale_body,
        out_shape=jax.ShapeDtypeStruct(W_v.shape, W_v.dtype),
    )(W_v)
    t3 = inputs.T.reshape(_L, _BT, 8)
    t4 = jnp.broadcast_to(t3[:, :, None, :], (_L, _BT, _H, 8))
    idx4 = t4.reshape(_L, _NW, _QW, _H * 8).transpose(1, 0, 2, 3)
    out4 = _sc_lookup(W_s, idx4)
    out = (out4.reshape(_L, _BT, _H, 8, _D)
           .transpose(1, 3, 0, 2, 4)
           .reshape(_B, _L, _H * _D))
    return out[None]


# speedup vs baseline: 1.2515x; 1.2515x over previous
import functools
import math

import jax
import jax.numpy as jnp
from jax import lax
from jax.experimental import pallas as pl
from jax.experimental.pallas import tpu as pltpu
from jax.experimental.pallas import tpu_sc as plsc

_D = 128
_H = 16
_SCALE = math.sqrt(float(_D))
_B = 1024
_L = 50
_NC = 2
_NS = 16
_NW = _NC * _NS
_BPW = _B // _NW


def _scale_body(w_ref, o_ref):
    o_ref[:] = w_ref[:] * _SCALE


_LB = 5
_NI = _L // _LB

_mesh = plsc.VectorSubcoreMesh(core_axis_name="c", subcore_axis_name="s")


@functools.partial(
    pl.kernel,
    out_type=jax.ShapeDtypeStruct((_L, _B, _H * _D), jnp.float32),
    mesh=_mesh,
    scratch_types=[
        pltpu.VMEM((_L, _BPW), jnp.int32),
        pltpu.VMEM((_LB, _BPW, _D), jnp.float32),
        pltpu.VMEM((_LB, _BPW, _D), jnp.float32),
        pltpu.VMEM((_LB, _BPW, _D), jnp.float32),
        pltpu.VMEM_SHARED((130, _D), jnp.float32),
        pltpu.SemaphoreType.DMA,
        pltpu.SemaphoreType.DMA,
        pltpu.SemaphoreType.DMA,
        pltpu.SemaphoreType.DMA,
    ],
    compiler_params=pltpu.CompilerParams(use_tc_tiling_on_sc=True),
)
def _sc_lookup(table_hbm, idx_hbm, out_hbm, idx_v, buf0, buf1, buf2, tshared,
               gsem, wsem0, wsem1, wsem2):
    wid = lax.axis_index("s") * _NC + lax.axis_index("c")
    b0 = wid * _BPW
    @pl.when(lax.axis_index("s") == 0)
    def _():
        pltpu.sync_copy(table_hbm, tshared)
    pltpu.sync_copy(idx_hbm.at[wid], idx_v)
    plsc.subcore_barrier()

    bufs = (buf0, buf1, buf2)
    wsems = (wsem0, wsem1, wsem2)
    nb = len(bufs)
    pending = [None] * nb

    cur_g = [pltpu.async_copy(tshared.at[idx_v.at[j]], buf0.at[j], gsem)
             for j in range(_LB)]
    for i in range(_NI):
        slot = i % nb
        for g in cur_g:
            g.wait()
        if i + 1 < _NI:
            nslot = (i + 1) % nb
            if pending[nslot] is not None:
                for w in pending[nslot]:
                    w.wait()
                pending[nslot] = None
            cur_g = [
                pltpu.async_copy(tshared.at[idx_v.at[(i + 1) * _LB + j]],
                                 bufs[nslot].at[j], gsem)
                for j in range(_LB)
            ]
        pending[slot] = [
            pltpu.async_copy(
                bufs[slot],
                out_hbm.at[pl.ds(i * _LB, _LB), pl.ds(b0, _BPW),
                           pl.ds(h * _D, _D)],
                wsems[slot])
            for h in range(_H)
        ]
    for ws in pending:
        if ws is not None:
            for w in ws:
                w.wait()


def kernel(inputs, W_v):
    W_s = pl.pallas_call(
        _scale_body,
        out_shape=jax.ShapeDtypeStruct(W_v.shape, W_v.dtype),
    )(W_v)
    idx3 = inputs.T.reshape(_L, _NW, _BPW).transpose(1, 0, 2)
    out = _sc_lookup(W_s, idx3)
    return jnp.transpose(out, (1, 0, 2))[None]


# speedup vs baseline: 1.2600x; 1.0068x over previous
import functools
import math

import jax
import jax.numpy as jnp
from jax import lax
from jax.experimental import pallas as pl
from jax.experimental.pallas import tpu as pltpu
from jax.experimental.pallas import tpu_sc as plsc

_D = 128
_H = 16
_SCALE = math.sqrt(float(_D))
_B = 1024
_L = 50
_NC = 2
_NS = 16
_NW = _NC * _NS
_BPW = _B // _NW


def _scale_body(w_ref, o_ref):
    o_ref[:] = w_ref[:] * _SCALE


_LB = 10
_NI = _L // _LB

_mesh = plsc.VectorSubcoreMesh(core_axis_name="c", subcore_axis_name="s")


@functools.partial(
    pl.kernel,
    out_type=jax.ShapeDtypeStruct((_L, _B, _H * _D), jnp.float32),
    mesh=_mesh,
    scratch_types=[
        pltpu.VMEM((_L, _BPW), jnp.int32),
        pltpu.VMEM((_LB, _BPW, _D), jnp.float32),
        pltpu.VMEM((_LB, _BPW, _D), jnp.float32),
        pltpu.VMEM_SHARED((130, _D), jnp.float32),
        pltpu.SemaphoreType.DMA,
        pltpu.SemaphoreType.DMA,
        pltpu.SemaphoreType.DMA,
    ],
    compiler_params=pltpu.CompilerParams(use_tc_tiling_on_sc=True),
)
def _sc_lookup(table_hbm, idx_hbm, out_hbm, idx_v, buf0, buf1, tshared,
               gsem, wsem0, wsem1):
    wid = lax.axis_index("s") * _NC + lax.axis_index("c")
    b0 = wid * _BPW
    @pl.when(lax.axis_index("s") == 0)
    def _():
        pltpu.sync_copy(table_hbm, tshared)
    pltpu.sync_copy(idx_hbm.at[wid], idx_v)
    plsc.subcore_barrier()

    bufs = (buf0, buf1)
    wsems = (wsem0, wsem1)
    pending = [None, None]

    cur_g = [pltpu.async_copy(tshared.at[idx_v.at[j]], buf0.at[j], gsem)
             for j in range(_LB)]
    for i in range(_NI):
        slot = i % 2
        for g in cur_g:
            g.wait()
        if i + 1 < _NI:
            nslot = (i + 1) % 2
            if pending[nslot] is not None:
                for w in pending[nslot]:
                    w.wait()
                pending[nslot] = None
            cur_g = [
                pltpu.async_copy(tshared.at[idx_v.at[(i + 1) * _LB + j]],
                                 bufs[nslot].at[j], gsem)
                for j in range(_LB)
            ]
        pending[slot] = [
            pltpu.async_copy(
                bufs[slot],
                out_hbm.at[pl.ds(i * _LB, _LB), pl.ds(b0, _BPW),
                           pl.ds(h * _D, _D)],
                wsems[slot])
            for h in range(_H)
        ]
    for ws in pending:
        if ws is not None:
            for w in ws:
                w.wait()


def kernel(inputs, W_v):
    W_s = pl.pallas_call(
        _scale_body,
        out_shape=jax.ShapeDtypeStruct(W_v.shape, W_v.dtype),
    )(W_v)
    idx3 = inputs.T.reshape(_L, _NW, _BPW).transpose(1, 0, 2)
    out = _sc_lookup(W_s, idx3)
    return jnp.transpose(out, (1, 0, 2))[None]
